# baseline (speedup 1.0000x reference)
import jax
import jax.numpy as jnp
from jax import lax
from jax.experimental import pallas as pl
from jax.experimental.pallas import tpu as pltpu
from jax.experimental.pallas import tpu_sc as plsc

B = 16384
K = 16
NC = 2
NS = 16
NW = NC * NS
BPW = B // NW
CHUNK = 128
NCHUNK = BPW // CHUNK
PACK = 128 // K


def _sc_body(uid_hbm, iid_hbm, uw_hbm, iw_hbm, out_hbm,
             uidx_v, iidx_v, cu_idx, ci_idx, cu_rows, ci_rows, outv,
             sems):
    wid = lax.axis_index("s") * NC + lax.axis_index("c")
    base = wid * BPW

    pltpu.sync_copy(uid_hbm.at[pl.ds(base, BPW)], uidx_v)
    pltpu.sync_copy(iid_hbm.at[pl.ds(base, BPW)], iidx_v)

    for t in range(BPW // K):
        j, c = t // (CHUNK // K), (t % (CHUNK // K)) * K
        cu_idx[j, pl.ds(c, K)] = lax.shift_right_logical(
            uidx_v[pl.ds(t * K, K)], 3)
        ci_idx[j, pl.ds(c, K)] = lax.shift_right_logical(
            iidx_v[pl.ds(t * K, K)], 3)

    def start(j):
        buf = j % 2
        cu = pltpu.async_copy(uw_hbm.at[cu_idx.at[j]], cu_rows.at[buf],
                              sems.at[buf, 0])
        ci = pltpu.async_copy(iw_hbm.at[ci_idx.at[j]], ci_rows.at[buf],
                              sems.at[buf, 1])
        return cu, ci

    iota = lax.iota(jnp.int32, K)
    pending = start(0)

    for j in range(NCHUNK):
        pending[0].wait()
        pending[1].wait()
        if j + 1 < NCHUNK:
            pending = start(j + 1)
        buf = j % 2
        for g in range(CHUNK // K):
            rows = g * K + iota
            uoff = (uidx_v[pl.ds(j * CHUNK + g * K, K)] & 7) * K
            ioff = (iidx_v[pl.ds(j * CHUNK + g * K, K)] & 7) * K
            acc = jnp.zeros((K,), jnp.float32)
            for k in range(K):
                uc = plsc.load_gather(cu_rows, [jnp.full((K,), buf, jnp.int32),
                                                rows, uoff + k])
                ic = plsc.load_gather(ci_rows, [jnp.full((K,), buf, jnp.int32),
                                                rows, ioff + k])
                acc = acc + uc * ic
            outv[pl.ds(j * CHUNK + g * K, K)] = acc

    pltpu.sync_copy(outv, out_hbm.at[pl.ds(base, BPW)])


@jax.jit
def kernel(train_x, user_weight, item_weight):
    uid = train_x[:, 0]
    iid = train_x[:, 1]
    uw = user_weight.reshape(-1, 128)
    iw = item_weight.reshape(-1, 128)

    mesh = plsc.VectorSubcoreMesh(
        core_axis_name="c", subcore_axis_name="s",
        num_cores=NC, num_subcores=NS)
    fn = pl.kernel(
        _sc_body,
        out_type=jax.ShapeDtypeStruct((B,), jnp.float32),
        mesh=mesh,
        scratch_types=[
            pltpu.VMEM((BPW,), jnp.int32),
            pltpu.VMEM((BPW,), jnp.int32),
            pltpu.VMEM((NCHUNK, CHUNK), jnp.int32),
            pltpu.VMEM((NCHUNK, CHUNK), jnp.int32),
            pltpu.VMEM((2, CHUNK, 128), jnp.float32),
            pltpu.VMEM((2, CHUNK, 128), jnp.float32),
            pltpu.VMEM((BPW,), jnp.float32),
            pltpu.SemaphoreType.DMA((2, 2)),
        ],
        compiler_params=pltpu.CompilerParams(needs_layout_passes=False),
    )
    return fn(uid, iid, uw, iw)


# speedup vs baseline: 5.9358x; 5.9358x over previous
import jax
import jax.numpy as jnp
from jax import lax
from jax.experimental import pallas as pl
from jax.experimental.pallas import tpu as pltpu
from jax.experimental.pallas import tpu_sc as plsc

B = 16384
K = 16
NC = 2
NS = 16
NW = NC * NS
BPW = B // NW
CH = 16
NCH = BPW // CH


def _sc_body(uid_hbm, iid_hbm, uwt_hbm, iwt_hbm, dummy_hbm, out_hbm,
             uidx_v, iidx_v, ublk, iblk, outv, sem_u, sem_i):
    wid = lax.axis_index("s") * NC + lax.axis_index("c")
    base = wid * BPW

    pltpu.sync_copy(uid_hbm.at[pl.ds(base, BPW)], uidx_v)
    pltpu.sync_copy(iid_hbm.at[pl.ds(base, BPW)], iidx_v)

    iota = lax.iota(jnp.int32, K)

    def chunk(c, carry):
        uvec = uidx_v[pl.ds(c * CH, CH)]
        ivec = iidx_v[pl.ds(c * CH, CH)]
        ucb = lax.shift_left(lax.shift_right_logical(uvec, 7), 7)
        icb = lax.shift_left(lax.shift_right_logical(ivec, 7), 7)
        for l in range(CH):
            uo = pl.multiple_of(ucb[l], 128)
            io = pl.multiple_of(icb[l], 128)
            pltpu.async_copy(uwt_hbm.at[:, pl.ds(uo, 128)],
                             ublk.at[l], sem_u)
            pltpu.async_copy(iwt_hbm.at[:, pl.ds(io, 128)],
                             iblk.at[l], sem_i)
        pltpu.make_async_copy(dummy_hbm, ublk, sem_u).wait()
        pltpu.make_async_copy(dummy_hbm, iblk, sem_i).wait()

        ulane = uvec & 127
        ilane = ivec & 127
        acc = jnp.zeros((K,), jnp.float32)
        for k in range(K):
            kv = jnp.full((K,), k, jnp.int32)
            uc = plsc.load_gather(ublk, [iota, kv, ulane])
            ic = plsc.load_gather(iblk, [iota, kv, ilane])
            acc = acc + uc * ic
        outv[pl.ds(c * CH, CH)] = acc
        return carry

    lax.fori_loop(0, NCH, chunk, 0)

    pltpu.sync_copy(outv, out_hbm.at[pl.ds(base, BPW)])


@jax.jit
def kernel(train_x, user_weight, item_weight):
    uid = train_x[:, 0]
    iid = train_x[:, 1]
    dummy = jnp.zeros((CH, K, 128), jnp.float32)

    mesh = plsc.VectorSubcoreMesh(
        core_axis_name="c", subcore_axis_name="s",
        num_cores=NC, num_subcores=NS)
    fn = pl.kernel(
        _sc_body,
        out_type=jax.ShapeDtypeStruct((B,), jnp.float32),
        mesh=mesh,
        scratch_types=[
            pltpu.VMEM((BPW,), jnp.int32),
            pltpu.VMEM((BPW,), jnp.int32),
            pltpu.VMEM((CH, K, 128), jnp.float32),
            pltpu.VMEM((CH, K, 128), jnp.float32),
            pltpu.VMEM((BPW,), jnp.float32),
            pltpu.SemaphoreType.DMA,
            pltpu.SemaphoreType.DMA,
        ],
        compiler_params=pltpu.CompilerParams(needs_layout_passes=False),
    )
    return fn(uid, iid, user_weight.T, item_weight.T, dummy)


# speedup vs baseline: 6.1717x; 1.0397x over previous
import jax
import jax.numpy as jnp
from jax import lax
from jax.experimental import pallas as pl
from jax.experimental.pallas import tpu as pltpu
from jax.experimental.pallas import tpu_sc as plsc

B = 16384
K = 16
NC = 2
NS = 16
NW = NC * NS
BPW = B // NW
CH = 8
NCH = BPW // CH


def _sc_body(uid_hbm, iid_hbm, uwt_hbm, iwt_hbm, dummy_hbm, out_hbm,
             uidx_v, iidx_v, blk, outv, sems):
    wid = lax.axis_index("s") * NC + lax.axis_index("c")
    base = wid * BPW

    pltpu.sync_copy(uid_hbm.at[pl.ds(base, BPW)], uidx_v)
    pltpu.sync_copy(iid_hbm.at[pl.ds(base, BPW)], iidx_v)

    iota = lax.iota(jnp.int32, 2 * CH)
    slot8 = iota & 7

    def issue(c):
        buf = lax.rem(c, 2)
        uvec = plsc.load_gather(uidx_v, [c * CH + slot8])
        ivec = plsc.load_gather(iidx_v, [c * CH + slot8])
        ucb = lax.shift_left(lax.shift_right_logical(uvec, 7), 7)
        icb = lax.shift_left(lax.shift_right_logical(ivec, 7), 7)
        for l in range(CH):
            uo = pl.multiple_of(ucb[l], 128)
            io = pl.multiple_of(icb[l], 128)
            pltpu.async_copy(uwt_hbm.at[:, pl.ds(uo, 128)],
                             blk.at[buf, l], sems.at[buf])
            pltpu.async_copy(iwt_hbm.at[:, pl.ds(io, 128)],
                             blk.at[buf, CH + l], sems.at[buf])

    def chunk(c, carry):
        buf = lax.rem(c, 2)

        @pl.when(c + 1 < NCH)
        def _():
            issue(c + 1)

        pltpu.make_async_copy(dummy_hbm, blk.at[buf], sems.at[buf]).wait()

        ulane = plsc.load_gather(uidx_v, [c * CH + slot8]) & 127
        ilane = plsc.load_gather(iidx_v, [c * CH + slot8]) & 127
        bufv = jnp.full((2 * CH,), buf, jnp.int32)
        acc = jnp.zeros((2 * CH,), jnp.float32)
        for k in range(K):
            kv = jnp.full((2 * CH,), k, jnp.int32)
            uc = plsc.load_gather(blk, [bufv, slot8, kv, ulane])
            ic = plsc.load_gather(blk, [bufv, slot8 + CH, kv, ilane])
            acc = acc + uc * ic
        outv[pl.ds(c * CH, 2 * CH)] = acc
        return carry

    issue(0)
    lax.fori_loop(0, NCH, chunk, 0)

    pltpu.sync_copy(outv.at[pl.ds(0, BPW)], out_hbm.at[pl.ds(base, BPW)])


@jax.jit
def kernel(train_x, user_weight, item_weight):
    uid = train_x[:, 0]
    iid = train_x[:, 1]
    dummy = jnp.zeros((2 * CH, K, 128), jnp.float32)

    mesh = plsc.VectorSubcoreMesh(
        core_axis_name="c", subcore_axis_name="s",
        num_cores=NC, num_subcores=NS)
    fn = pl.kernel(
        _sc_body,
        out_type=jax.ShapeDtypeStruct((B,), jnp.float32),
        mesh=mesh,
        scratch_types=[
            pltpu.VMEM((BPW,), jnp.int32),
            pltpu.VMEM((BPW,), jnp.int32),
            pltpu.VMEM((2, 2 * CH, K, 128), jnp.float32),
            pltpu.VMEM((BPW + CH,), jnp.float32),
            pltpu.SemaphoreType.DMA((2,)),
        ],
        compiler_params=pltpu.CompilerParams(needs_layout_passes=False),
    )
    return fn(uid, iid, user_weight.T, item_weight.T, dummy)
